# baseline (speedup 1.0000x reference)
import jax
import jax.numpy as jnp
from jax import lax
from jax.experimental import pallas as pl
from jax.experimental.pallas import tpu as pltpu
from jax.experimental.pallas import tpu_sc as plsc

EMBED = 8
LANES = 16
NUM_CORES = 2
NUM_SUBCORES = 16
NUM_WORKERS = NUM_CORES * NUM_SUBCORES


def _dot_body(chunk, groups, users_hbm, items_hbm, ut_hbm, it_hbm, out_hbm,
              idx_u, idx_i, rows_u, rows_i, out_v, sem_u, sem_i):
    wid = lax.axis_index("s") * NUM_CORES + lax.axis_index("c")
    base = wid * chunk
    pltpu.sync_copy(users_hbm.at[pl.ds(base, chunk)], idx_u)
    pltpu.sync_copy(items_hbm.at[pl.ds(base, chunk)], idx_i)
    cp_u = pltpu.async_copy(ut_hbm.at[idx_u], rows_u, sem_u)
    cp_i = pltpu.async_copy(it_hbm.at[idx_i], rows_i, sem_i)
    cp_u.wait()
    cp_i.wait()
    lane = lax.iota(jnp.int32, LANES)

    def group(g, carry):
        row = lane + g * LANES
        acc = None
        for d in range(EMBED):
            col = jnp.full((LANES,), d, jnp.int32)
            u = plsc.load_gather(rows_u, [row, col])
            v = plsc.load_gather(rows_i, [row, col])
            acc = u * v if acc is None else acc + u * v
        out_v[pl.ds(g * LANES, LANES)] = acc
        return carry

    lax.fori_loop(0, groups, group, 0)
    pltpu.sync_copy(out_v, out_hbm.at[pl.ds(base, chunk)])


def kernel(users, items, user_table, item_table):
    batch = users.shape[0]
    chunk = batch // NUM_WORKERS
    groups = chunk // LANES
    mesh = plsc.VectorSubcoreMesh(core_axis_name="c", subcore_axis_name="s")

    def body(*refs):
        _dot_body(chunk, groups, *refs)

    k = pl.kernel(
        body,
        mesh=mesh,
        compiler_params=pltpu.CompilerParams(
            needs_layout_passes=False, use_tc_tiling_on_sc=False),
        out_type=jax.ShapeDtypeStruct((batch,), jnp.float32),
        scratch_types=[
            pltpu.VMEM((chunk,), jnp.int32),
            pltpu.VMEM((chunk,), jnp.int32),
            pltpu.VMEM((chunk, EMBED), jnp.float32),
            pltpu.VMEM((chunk, EMBED), jnp.float32),
            pltpu.VMEM((chunk,), jnp.float32),
            pltpu.SemaphoreType.DMA,
            pltpu.SemaphoreType.DMA,
        ],
    )
    return k(users.astype(jnp.int32), items.astype(jnp.int32),
             user_table, item_table)
